# baseline (speedup 1.0000x reference)
import functools

import jax
import jax.numpy as jnp
from jax import lax
from jax.experimental import pallas as pl
from jax.experimental.pallas import tpu as pltpu
from jax.experimental.pallas import tpu_sc as plsc

D = 128
N_WORKERS = 32
B = 128


def _sqrt16(q):
    qi = lax.bitcast_convert_type(q, jnp.int32)
    yi = 0x5F3759DF - (qi >> 1)
    y = lax.bitcast_convert_type(yi, jnp.float32)
    for _ in range(3):
        y = y * (1.5 - 0.5 * q * y * y)
    return q * y


@functools.lru_cache(maxsize=None)
def _make_kernel(NB: int):
    C = NB * B
    mesh = plsc.VectorSubcoreMesh(core_axis_name="c", subcore_axis_name="s")

    @functools.partial(
        pl.kernel,
        mesh=mesh,
        compiler_params=pltpu.CompilerParams(needs_layout_passes=False),
        out_type=jax.ShapeDtypeStruct((N_WORKERS, 16), jnp.float32),
        scratch_types=[
            pltpu.VMEM((B,), jnp.int32),
            pltpu.VMEM((B,), jnp.int32),
            pltpu.VMEM((B,), jnp.float32),
            pltpu.VMEM((B, D), jnp.float32),
            pltpu.VMEM((B, D), jnp.float32),
            pltpu.VMEM((16,), jnp.float32),
            pltpu.SemaphoreType.DMA,
            pltpu.SemaphoreType.DMA,
        ],
    )
    def k(x_hbm, idx0_hbm, idx1_hbm, s_hbm, out_hbm,
          idxi_v, idxj_v, s_v, rows_i, rows_j, loss_v, sem_i, sem_j):
        cid = lax.axis_index("c")
        sid = lax.axis_index("s")
        wid = sid * 2 + cid
        base = wid * C

        def block_body(b, loss):
            off = base + b * B
            pltpu.sync_copy(idx0_hbm.at[pl.ds(off, B)], idxi_v)
            pltpu.sync_copy(idx1_hbm.at[pl.ds(off, B)], idxj_v)
            pltpu.sync_copy(s_hbm.at[pl.ds(off, B)], s_v)
            cp_i = pltpu.async_copy(x_hbm.at[idxi_v], rows_i, sem_i)
            cp_j = pltpu.async_copy(x_hbm.at[idxj_v], rows_j, sem_j)
            cp_i.wait()
            cp_j.wait()
            for g in range(B // 16):
                row16 = lax.iota(jnp.int32, 16) + g * 16

                def f_body(f, acc):
                    col = jnp.full((16,), f, jnp.int32)
                    vi = plsc.load_gather(rows_i, [row16, col])
                    vj = plsc.load_gather(rows_j, [row16, col])
                    t = vi - vj
                    return acc + t * t

                q = lax.fori_loop(0, D, f_body, jnp.zeros((16,), jnp.float32))
                d = _sqrt16(q)
                sv = s_v[pl.ds(g * 16, 16)]
                t = sv - d
                loss = loss + t * t
            return loss

        loss = lax.fori_loop(0, NB, block_body, jnp.zeros((16,), jnp.float32))
        loss_v[...] = loss
        pltpu.sync_copy(loss_v, out_hbm.at[wid])

    return k


def kernel(input, small_dists, indices):
    E = indices.shape[0]
    NB = -(-E // (N_WORKERS * B))
    E_pad = N_WORKERS * B * NB
    pad = E_pad - E
    idx0 = jnp.pad(indices[:, 0], (0, pad))
    idx1 = jnp.pad(indices[:, 1], (0, pad))
    s = jnp.pad(small_dists, (0, pad))
    out = _make_kernel(NB)(input, idx0, idx1, s)
    return out.sum()


# speedup vs baseline: 2.6598x; 2.6598x over previous
import functools

import jax
import jax.numpy as jnp
from jax import lax
from jax.experimental import pallas as pl
from jax.experimental.pallas import tpu as pltpu
from jax.experimental.pallas import tpu_sc as plsc

D = 128
N_WORKERS = 32
B = 128


@functools.lru_cache(maxsize=None)
def _make_kernel(NB: int):
    C = NB * B
    mesh = plsc.VectorSubcoreMesh(core_axis_name="c", subcore_axis_name="s")

    @functools.partial(
        pl.kernel,
        mesh=mesh,
        compiler_params=pltpu.CompilerParams(needs_layout_passes=False),
        out_type=jax.ShapeDtypeStruct((N_WORKERS, 16), jnp.float32),
        scratch_types=[
            pltpu.VMEM((C,), jnp.int32),
            pltpu.VMEM((C,), jnp.int32),
            pltpu.VMEM((C,), jnp.float32),
            pltpu.VMEM((B, D), jnp.float32),
            pltpu.VMEM((B, D), jnp.float32),
            pltpu.VMEM((B, D), jnp.float32),
            pltpu.VMEM((B, D), jnp.float32),
            pltpu.VMEM((16,), jnp.float32),
            pltpu.SemaphoreType.DMA,
            pltpu.SemaphoreType.DMA,
        ],
    )
    def k(x_hbm, idx0_hbm, idx1_hbm, s_hbm, out_hbm,
          idxi_v, idxj_v, s_v, riA, rjA, riB, rjB, loss_v, semA, semB):
        cid = lax.axis_index("c")
        sid = lax.axis_index("s")
        wid = sid * 2 + cid
        base = wid * C

        pltpu.sync_copy(idx0_hbm.at[pl.ds(base, C)], idxi_v)
        pltpu.sync_copy(idx1_hbm.at[pl.ds(base, C)], idxj_v)
        pltpu.sync_copy(s_hbm.at[pl.ds(base, C)], s_v)

        def issue(b, ri, rj, sem):
            off = jnp.where(b >= NB, 0, b * B)
            pltpu.async_copy(x_hbm.at[idxi_v.at[pl.ds(off, B)]], ri, sem)
            pltpu.async_copy(x_hbm.at[idxj_v.at[pl.ds(off, B)]], rj, sem)

        def drain(ri, rj, sem):
            pltpu.make_async_copy(x_hbm.at[idxi_v.at[pl.ds(0, B)]], ri, sem).wait()
            pltpu.make_async_copy(x_hbm.at[idxj_v.at[pl.ds(0, B)]], rj, sem).wait()

        def compute(ri, rj, b, loss2):
            def grp(g, loss2):
                l0, l1 = loss2
                sv = s_v[pl.ds(b * B + g * 16, 16)]
                row = g * 16
                for l in range(16):
                    e = row + l
                    t0 = ri[e, pl.ds(0, 16)] - rj[e, pl.ds(0, 16)]
                    acc = t0 * t0
                    for c in range(1, D // 16):
                        tc = ri[e, pl.ds(c * 16, 16)] - rj[e, pl.ds(c * 16, 16)]
                        acc = acc + tc * tc
                    q = jnp.sum(acc)
                    qi = lax.bitcast_convert_type(q, jnp.int32)
                    yi = jnp.int32(0x5F3759DF) - (qi >> 1)
                    y = lax.bitcast_convert_type(yi, jnp.float32)
                    for _ in range(3):
                        y = y * (1.5 - 0.5 * q * y * y)
                    d = q * y
                    t = sv[l] - d
                    if l % 2 == 0:
                        l0 = l0 + t * t
                    else:
                        l1 = l1 + t * t
                return l0, l1
            return lax.fori_loop(0, B // 16, grp, loss2)

        issue(0, riA, rjA, semA)

        def outer(bb, loss2):
            b0 = 2 * bb
            drain(riA, rjA, semA)
            issue(b0 + 1, riB, rjB, semB)
            loss2 = compute(riA, rjA, b0, loss2)
            drain(riB, rjB, semB)
            issue(b0 + 2, riA, rjA, semA)
            loss2 = compute(riB, rjB, b0 + 1, loss2)
            return loss2

        l0, l1 = lax.fori_loop(
            0, NB // 2, outer, (jnp.float32(0.0), jnp.float32(0.0)))
        drain(riA, rjA, semA)
        lane = lax.iota(jnp.int32, 16)
        loss_v[...] = jnp.where(lane == 0, l0 + l1, 0.0)
        pltpu.sync_copy(loss_v, out_hbm.at[wid])

    return k


def kernel(input, small_dists, indices):
    E = indices.shape[0]
    NB = 2 * -(-E // (N_WORKERS * B * 2))
    E_pad = N_WORKERS * B * NB
    pad = E_pad - E
    idx0 = jnp.pad(indices[:, 0], (0, pad))
    idx1 = jnp.pad(indices[:, 1], (0, pad))
    s = jnp.pad(small_dists, (0, pad))
    out = _make_kernel(NB)(input, idx0, idx1, s)
    return out.sum()


# speedup vs baseline: 6.8696x; 2.5827x over previous
import functools

import jax
import jax.numpy as jnp
from jax import lax
from jax.experimental import pallas as pl
from jax.experimental.pallas import tpu as pltpu
from jax.experimental.pallas import tpu_sc as plsc

N = 8192
D = 128
N_WORKERS = 32
ROWS_W = N // N_WORKERS
B = 128
SB = 64


def _newton_sqrt(q):
    qi = lax.bitcast_convert_type(q, jnp.int32)
    yi = 0x5F3759DF - (qi >> 1)
    y = lax.bitcast_convert_type(yi, jnp.float32)
    for _ in range(3):
        y = y * (1.5 - 0.5 * q * y * y)
    return q * y


@functools.lru_cache(maxsize=None)
def _make_kernel(L: int):
    NBLK = L // B
    mesh = plsc.VectorSubcoreMesh(core_axis_name="c", subcore_axis_name="s")

    @functools.partial(
        pl.kernel,
        mesh=mesh,
        compiler_params=pltpu.CompilerParams(needs_layout_passes=False),
        out_type=jax.ShapeDtypeStruct((N_WORKERS, 16), jnp.float32),
        scratch_types=[
            pltpu.VMEM((ROWS_W, D), jnp.float32),
            pltpu.VMEM((SB * B,), jnp.int32),
            pltpu.VMEM((SB * B,), jnp.int32),
            pltpu.VMEM((SB * B,), jnp.float32),
            pltpu.VMEM((B, D), jnp.float32),
            pltpu.VMEM((B, D), jnp.float32),
            pltpu.VMEM((16 * 17,), jnp.float32),
            pltpu.VMEM((16,), jnp.int32),
            pltpu.VMEM((16,), jnp.float32),
            pltpu.SemaphoreType.DMA,
            pltpu.SemaphoreType.DMA,
        ],
    )
    def k(x_hbm, idx0_hbm, idx1_hbm, s_hbm, bnd_hbm, out_hbm,
          xi_v, idx0_v, idx1_v, s_v, rjA, rjB, tb_v, bnd_v, loss_v,
          semA, semB):
        cid = lax.axis_index("c")
        sid = lax.axis_index("s")
        wid = sid * 2 + cid
        base_node = wid * ROWS_W

        pltpu.sync_copy(bnd_hbm.at[wid], bnd_v)
        bnd = bnd_v[...]
        lo = bnd[0]
        hi = bnd[1]
        blk0 = lo // B
        blk_end = (hi + B - 1) // B

        pltpu.sync_copy(x_hbm.at[pl.ds(base_node, ROWS_W)], xi_v)

        lane = lax.iota(jnp.int32, 16)
        t_idx0 = lane * 17

        def issue(local_b, rj, sem):
            lb = jnp.where(local_b >= SB, 0, local_b)
            pltpu.async_copy(
                x_hbm.at[idx1_v.at[pl.ds(lb * B, B)]], rj, sem)

        def drain(rj, sem):
            pltpu.make_async_copy(
                x_hbm.at[idx1_v.at[pl.ds(0, B)]], rj, sem).wait()

        def compute(rj, cblk0, local_b, loss16):
            gb = cblk0 + local_b
            eb = gb * B

            def grp(g, loss16):
                soff = local_b * B + g * 16
                sv = s_v[pl.ds(soff, 16)]
                iv = idx0_v[pl.ds(soff, 16)]
                for l in range(16):
                    il = iv[l] - base_node
                    il = jnp.minimum(jnp.maximum(il, 0), ROWS_W - 1)
                    e = g * 16 + l
                    t0 = xi_v[il, pl.ds(0, 16)] - rj[e, pl.ds(0, 16)]
                    a0 = t0 * t0
                    t1 = xi_v[il, pl.ds(16, 16)] - rj[e, pl.ds(16, 16)]
                    a1 = t1 * t1
                    for c in range(2, D // 16, 2):
                        tc = xi_v[il, pl.ds(c * 16, 16)] - rj[e, pl.ds(c * 16, 16)]
                        a0 = a0 + tc * tc
                        td = xi_v[il, pl.ds(c * 16 + 16, 16)] - rj[e, pl.ds(c * 16 + 16, 16)]
                        a1 = a1 + td * td
                    tb_v[pl.ds(l * 17, 16)] = a0 + a1
                q = plsc.load_gather(tb_v, [t_idx0])
                for f in range(1, 16):
                    q = q + plsc.load_gather(tb_v, [t_idx0 + f])
                d = _newton_sqrt(q)
                t = sv - d
                e16 = eb + g * 16 + lane
                m = jnp.logical_and(e16 >= lo, e16 < hi)
                return loss16 + jnp.where(m, t * t, 0.0)

            return lax.fori_loop(0, B // 16, grp, loss16)

        def chunk_body(c, loss16):
            cblk0 = blk0 + c * SB
            soff = cblk0 * B
            pltpu.sync_copy(idx0_hbm.at[pl.ds(soff, SB * B)], idx0_v)
            pltpu.sync_copy(idx1_hbm.at[pl.ds(soff, SB * B)], idx1_v)
            pltpu.sync_copy(s_hbm.at[pl.ds(soff, SB * B)], s_v)
            npairs = jnp.minimum(SB, blk_end - cblk0)
            npairs = (npairs + 1) // 2

            issue(0, rjA, semA)

            def pair(p, loss16):
                a = 2 * p
                drain(rjA, semA)
                issue(a + 1, rjB, semB)
                loss16 = compute(rjA, cblk0, a, loss16)
                drain(rjB, semB)
                issue(a + 2, rjA, semA)
                loss16 = compute(rjB, cblk0, a + 1, loss16)
                return loss16

            loss16 = lax.fori_loop(0, npairs, pair, loss16)
            drain(rjA, semA)
            return loss16

        nchunks = (blk_end - blk0 + SB - 1) // SB
        loss16 = lax.fori_loop(
            0, nchunks, chunk_body, jnp.zeros((16,), jnp.float32))
        loss_v[...] = loss16
        pltpu.sync_copy(loss_v, out_hbm.at[wid])

    return k


def kernel(input, small_dists, indices):
    E = indices.shape[0]
    L = -(-E // B) * B + SB * B
    pad = L - E
    idx0 = jnp.pad(indices[:, 0], (0, pad))
    idx1 = jnp.pad(indices[:, 1], (0, pad))
    s = jnp.pad(small_dists, (0, pad))
    cuts = jnp.arange(N_WORKERS + 1, dtype=jnp.int32) * ROWS_W
    b = jnp.searchsorted(indices[:, 0], cuts, side="left").astype(jnp.int32)
    bnd = jnp.zeros((N_WORKERS, 16), jnp.int32)
    bnd = bnd.at[:, 0].set(b[:-1]).at[:, 1].set(b[1:])
    out = _make_kernel(L)(input, idx0, idx1, s, bnd)
    return out.sum()
